# baseline (speedup 1.0000x reference)
import functools

import jax
import jax.numpy as jnp
from jax import lax
from jax.experimental import pallas as pl
from jax.experimental.pallas import tpu as pltpu
from jax.experimental.pallas import tpu_sc as plsc

N_NODES = 10000
N_EDGES = 320000
N_GRAPHS = 16

NPAD = 10240
EPAD = 327680
CHUNK = 128
ROWS_PER_SUB = NPAD // 16

M = 8
L = 4

_mesh = plsc.VectorSubcoreMesh(core_axis_name="c", subcore_axis_name="s")


def _sc_deg_body(dst_hbm, zeros_hbm, ones_hbm, out_hbm, dst_v, ones_v, sem,
                 shared):
    cid = lax.axis_index("c")
    sid = lax.axis_index("s")
    wid = cid * 16 + sid
    n_chunks = EPAD // 32 // CHUNK

    pltpu.sync_copy(zeros_hbm.at[pl.ds(sid * ROWS_PER_SUB, ROWS_PER_SUB)],
                    shared.at[pl.ds(sid * ROWS_PER_SUB, ROWS_PER_SUB)])
    pltpu.sync_copy(dst_hbm.at[wid], dst_v)
    pltpu.sync_copy(ones_hbm, ones_v)
    plsc.subcore_barrier()

    def fire(c, carry):
        pltpu.async_copy(ones_v, shared.at[dst_v.at[c]], sem, add=True)
        return carry

    lax.fori_loop(0, n_chunks, fire, 0)

    def drain(c, carry):
        pltpu.make_async_copy(ones_v, shared.at[dst_v.at[c]], sem).wait()
        return carry

    lax.fori_loop(0, n_chunks, drain, 0)
    plsc.subcore_barrier()
    pltpu.sync_copy(
        shared.at[pl.ds(sid * ROWS_PER_SUB, ROWS_PER_SUB), pl.ds(0, 8)],
        out_hbm.at[cid, pl.ds(sid * ROWS_PER_SUB, ROWS_PER_SUB)])


_sc_deg = pl.kernel(
    _sc_deg_body,
    out_type=jax.ShapeDtypeStruct((2, NPAD, 8), jnp.float32),
    mesh=_mesh,
    scratch_types=[
        pltpu.VMEM((EPAD // 32 // CHUNK, CHUNK), jnp.int32),
        pltpu.VMEM((CHUNK, 16), jnp.float32),
        pltpu.SemaphoreType.DMA,
        pltpu.VMEM_SHARED((NPAD, 16), jnp.float32),
    ],
    compiler_params=pltpu.CompilerParams(use_tc_tiling_on_sc=False),
)


def _sc_mp_pipeline(table, src_v, dst_v, rows_v, gsem, ssem, shared,
                    n_chunks, n_phases, load_idx):
    def fire_g(c, b):
        pltpu.async_copy(table.at[src_v.at[c]], rows_v.at[b], gsem.at[b])

    def wait_g(c, b):
        pltpu.make_async_copy(table.at[src_v.at[c]], rows_v.at[b],
                              gsem.at[b]).wait()

    def fire_s(c, b):
        pltpu.async_copy(rows_v.at[b], shared.at[dst_v.at[c]], ssem.at[b],
                         add=True)

    def wait_s(c, b):
        pltpu.make_async_copy(rows_v.at[b], shared.at[dst_v.at[c]],
                              ssem.at[b]).wait()

    def visit(c, b, do_drain, do_fire):
        wait_g(c, b)
        fire_s(c, b)
        cp = c + L
        if do_drain:
            wait_s(cp - M, cp % M)
        if do_fire:
            fire_g(cp, cp % M)

    for p in range(n_phases):
        load_idx(p)

        for b in range(L):
            fire_g(b, b)
        for b in range(M):
            visit(b, b, do_drain=(b >= L), do_fire=True)

        def group(g, carry):
            for b in range(M):
                visit(g * M + b, b, do_drain=True, do_fire=True)
            return carry

        lax.fori_loop(1, n_chunks // M - 1, group, 0)
        for b in range(M):
            c = n_chunks - M + b
            visit(c, b, do_drain=True, do_fire=(b < M - L))
        for b in range(L):
            c = n_chunks - L + b
            wait_s(c, c % M)


def _sc_mp1_body(hs_hbm, src_hbm, dst_hbm, zeros_hbm, out_hbm,
                 src_v, dst_v, rows_v, gsem, ssem, shared):
    cid = lax.axis_index("c")
    sid = lax.axis_index("s")
    wid = cid * 16 + sid
    n_chunks = EPAD // 32 // CHUNK

    pltpu.sync_copy(zeros_hbm.at[pl.ds(sid * ROWS_PER_SUB, ROWS_PER_SUB)],
                    shared.at[pl.ds(sid * ROWS_PER_SUB, ROWS_PER_SUB)])

    def load_idx(p):
        pltpu.sync_copy(src_hbm.at[wid], src_v)
        pltpu.sync_copy(dst_hbm.at[wid], dst_v)
        plsc.subcore_barrier()

    _sc_mp_pipeline(hs_hbm, src_v, dst_v, rows_v, gsem, ssem, shared,
                    n_chunks, 1, load_idx)

    plsc.subcore_barrier()
    pltpu.sync_copy(shared.at[pl.ds(sid * ROWS_PER_SUB, ROWS_PER_SUB)],
                    out_hbm.at[cid, pl.ds(sid * ROWS_PER_SUB, ROWS_PER_SUB)])


_sc_mp1 = pl.kernel(
    _sc_mp1_body,
    out_type=jax.ShapeDtypeStruct((2, NPAD, 64), jnp.bfloat16),
    mesh=_mesh,
    scratch_types=[
        pltpu.VMEM((EPAD // 32 // CHUNK, CHUNK), jnp.int32),
        pltpu.VMEM((EPAD // 32 // CHUNK, CHUNK), jnp.int32),
        pltpu.VMEM((M, CHUNK, 64), jnp.bfloat16),
        pltpu.SemaphoreType.DMA((M,)),
        pltpu.SemaphoreType.DMA((M,)),
        pltpu.VMEM_SHARED((NPAD, 64), jnp.bfloat16),
    ],
    compiler_params=pltpu.CompilerParams(use_tc_tiling_on_sc=False),
)


N_PHASES = 2
PHASE_CHUNKS = EPAD // 16 // CHUNK // N_PHASES


def _sc_mp2_body(hs_hbm, src_hbm, dst_hbm, zeros_hbm, out_hbm,
                 src_v, dst_v, rows_v, gsem, ssem, shared):
    cid = lax.axis_index("c")
    sid = lax.axis_index("s")
    table = hs_hbm.at[cid]

    pltpu.sync_copy(zeros_hbm.at[pl.ds(sid * ROWS_PER_SUB, ROWS_PER_SUB)],
                    shared.at[pl.ds(sid * ROWS_PER_SUB, ROWS_PER_SUB)])

    def load_idx(p):
        pltpu.sync_copy(src_hbm.at[2 * sid + p], src_v)
        pltpu.sync_copy(dst_hbm.at[2 * sid + p], dst_v)
        plsc.subcore_barrier()

    _sc_mp_pipeline(table, src_v, dst_v, rows_v, gsem, ssem, shared,
                    PHASE_CHUNKS, N_PHASES, load_idx)

    plsc.subcore_barrier()
    pltpu.sync_copy(shared.at[pl.ds(sid * ROWS_PER_SUB, ROWS_PER_SUB)],
                    out_hbm.at[cid, pl.ds(sid * ROWS_PER_SUB, ROWS_PER_SUB)])


_sc_mp2 = pl.kernel(
    _sc_mp2_body,
    out_type=jax.ShapeDtypeStruct((2, NPAD, 64), jnp.bfloat16),
    mesh=_mesh,
    scratch_types=[
        pltpu.VMEM((PHASE_CHUNKS, CHUNK), jnp.int32),
        pltpu.VMEM((PHASE_CHUNKS, CHUNK), jnp.int32),
        pltpu.VMEM((M, CHUNK, 64), jnp.bfloat16),
        pltpu.SemaphoreType.DMA((M,)),
        pltpu.SemaphoreType.DMA((M,)),
        pltpu.VMEM_SHARED((NPAD, 64), jnp.bfloat16),
    ],
    compiler_params=pltpu.CompilerParams(use_tc_tiling_on_sc=False),
)


def _dot1_body(x_ref, w1_ref, h_ref):
    h_ref[pl.ds(0, N_NODES), :] = jnp.dot(
        x_ref[...], w1_ref[...], preferred_element_type=jnp.float32)
    h_ref[pl.ds(N_NODES, NPAD - N_NODES), :] = jnp.zeros(
        (NPAD - N_NODES, 64), jnp.float32)


def _scale1_body(h_ref, degp_ref, hs1_ref, dinv_ref):
    degp = degp_ref[...]
    deg = 1.0 + degp[0, :, 0:1] + degp[1, :, 0:1]
    dinv = lax.rsqrt(deg)
    hs1_ref[...] = (h_ref[...] * dinv).astype(jnp.bfloat16)
    dinv_ref[...] = dinv


def _mm2_body(p1_ref, hs1_ref, dinv_ref, b1_ref, w2_ref, hs2_ref):
    p1 = p1_ref[...].astype(jnp.float32)
    hs1 = hs1_ref[...].astype(jnp.float32)
    dinv = dinv_ref[...]
    z = jax.nn.relu(dinv * (p1[0] + p1[1] + hs1) + b1_ref[...])
    hs2 = ((jnp.dot(z, w2_ref[...], preferred_element_type=jnp.float32))
           * dinv).astype(jnp.bfloat16)
    hs2_ref[0] = hs2[:, :64]
    hs2_ref[1] = hs2[:, 64:]


def _pool_body(p2_ref, hs2_ref, dinv_ref, b2_ref, batch_ref, out_ref):
    p2 = p2_ref[...].astype(jnp.float32)
    hs2 = hs2_ref[...].astype(jnp.float32)
    h2 = dinv_ref[...] * jnp.concatenate(
        [p2[0] + hs2[0], p2[1] + hs2[1]], axis=1)
    gids = lax.broadcasted_iota(jnp.int32, (N_GRAPHS, NPAD), 0)
    onehot = (gids == batch_ref[...]).astype(jnp.float32)
    pooled = jnp.dot(onehot, h2, preferred_element_type=jnp.float32)
    counts = jnp.maximum(jnp.sum(onehot, axis=1, keepdims=True), 1.0)
    out_ref[...] = pooled / counts + b2_ref[...]


_tc_dot1 = pl.pallas_call(
    _dot1_body,
    out_shape=jax.ShapeDtypeStruct((NPAD, 64), jnp.float32),
)

_tc_scale1 = pl.pallas_call(
    _scale1_body,
    out_shape=(jax.ShapeDtypeStruct((NPAD, 64), jnp.bfloat16),
               jax.ShapeDtypeStruct((NPAD, 1), jnp.float32)),
)

_tc_mm2 = pl.pallas_call(
    _mm2_body,
    out_shape=jax.ShapeDtypeStruct((2, NPAD, 64), jnp.bfloat16),
)

_tc_pool = pl.pallas_call(
    _pool_body,
    out_shape=jax.ShapeDtypeStruct((N_GRAPHS, 128), jnp.float32),
)


@jax.jit
def kernel(x, edge_index, batch, W1, b1, W2, b2):
    ei = edge_index.astype(jnp.int32)
    pad = N_NODES + (jnp.arange(EPAD - N_EDGES, dtype=jnp.int32)
                     % (NPAD - N_NODES))
    srcp = jnp.concatenate([ei[0], pad])
    dstp = jnp.concatenate([ei[1], pad])
    src32 = srcp.reshape(32, EPAD // 32 // CHUNK, CHUNK)
    dst32 = dstp.reshape(32, EPAD // 32 // CHUNK, CHUNK)
    batchp = jnp.pad(batch.astype(jnp.int32), (0, NPAD - N_NODES),
                     constant_values=-1).reshape(1, NPAD)

    zeros16 = jnp.zeros((NPAD, 16), jnp.float32)
    zeros64 = jnp.zeros((NPAD, 64), jnp.bfloat16)
    ones16 = jnp.ones((CHUNK, 16), jnp.float32)

    h = _tc_dot1(x.astype(jnp.float32), W1)
    degp = _sc_deg(dst32, zeros16, ones16)
    hs1, dinv = _tc_scale1(h, degp)
    p1 = _sc_mp1(hs1, src32, dst32, zeros64)
    hs2 = _tc_mm2(p1, hs1, dinv, b1.reshape(1, 64), W2)
    p2 = _sc_mp2(hs2, src32, dst32, zeros64)
    return _tc_pool(p2, hs2, dinv, b2.reshape(1, 128), batchp)


# speedup vs baseline: 1.0032x; 1.0032x over previous
import functools

import jax
import jax.numpy as jnp
from jax import lax
from jax.experimental import pallas as pl
from jax.experimental.pallas import tpu as pltpu
from jax.experimental.pallas import tpu_sc as plsc

N_NODES = 10000
N_EDGES = 320000
N_GRAPHS = 16

NPAD = 10112
EPAD = 327680
CHUNK = 128
ROWS_PER_SUB = NPAD // 16

M = 8
L = 4

_mesh = plsc.VectorSubcoreMesh(core_axis_name="c", subcore_axis_name="s")


def _sc_deg_body(dst_hbm, zeros_hbm, ones_hbm, out_hbm, dst_v, ones_v, sem,
                 shared):
    cid = lax.axis_index("c")
    sid = lax.axis_index("s")
    wid = cid * 16 + sid
    n_chunks = EPAD // 32 // CHUNK

    pltpu.sync_copy(zeros_hbm.at[pl.ds(sid * ROWS_PER_SUB, ROWS_PER_SUB)],
                    shared.at[pl.ds(sid * ROWS_PER_SUB, ROWS_PER_SUB)])
    pltpu.sync_copy(dst_hbm.at[wid], dst_v)
    pltpu.sync_copy(ones_hbm, ones_v)
    plsc.subcore_barrier()

    def fire(c, carry):
        pltpu.async_copy(ones_v, shared.at[dst_v.at[c]], sem, add=True)
        return carry

    lax.fori_loop(0, n_chunks, fire, 0)

    def drain(c, carry):
        pltpu.make_async_copy(ones_v, shared.at[dst_v.at[c]], sem).wait()
        return carry

    lax.fori_loop(0, n_chunks, drain, 0)
    plsc.subcore_barrier()
    pltpu.sync_copy(
        shared.at[pl.ds(sid * ROWS_PER_SUB, ROWS_PER_SUB), pl.ds(0, 8)],
        out_hbm.at[cid, pl.ds(sid * ROWS_PER_SUB, ROWS_PER_SUB)])


_sc_deg = pl.kernel(
    _sc_deg_body,
    out_type=jax.ShapeDtypeStruct((2, NPAD, 8), jnp.float32),
    mesh=_mesh,
    scratch_types=[
        pltpu.VMEM((EPAD // 32 // CHUNK, CHUNK), jnp.int32),
        pltpu.VMEM((CHUNK, 16), jnp.float32),
        pltpu.SemaphoreType.DMA,
        pltpu.VMEM_SHARED((NPAD, 16), jnp.float32),
    ],
    compiler_params=pltpu.CompilerParams(use_tc_tiling_on_sc=False),
)


def _sc_mp_pipeline(table, src_v, dst_v, rows_v, gsem, ssem, shared,
                    n_chunks, n_phases, load_idx):
    def fire_g(c, b):
        pltpu.async_copy(table.at[src_v.at[c]], rows_v.at[b], gsem.at[b])

    def wait_g(c, b):
        pltpu.make_async_copy(table.at[src_v.at[c]], rows_v.at[b],
                              gsem.at[b]).wait()

    def fire_s(c, b):
        pltpu.async_copy(rows_v.at[b], shared.at[dst_v.at[c]], ssem.at[b],
                         add=True)

    def wait_s(c, b):
        pltpu.make_async_copy(rows_v.at[b], shared.at[dst_v.at[c]],
                              ssem.at[b]).wait()

    def visit(c, b, do_drain, do_fire):
        wait_g(c, b)
        fire_s(c, b)
        cp = c + L
        if do_drain:
            wait_s(cp - M, cp % M)
        if do_fire:
            fire_g(cp, cp % M)

    for p in range(n_phases):
        load_idx(p)

        for b in range(L):
            fire_g(b, b)
        for b in range(M):
            visit(b, b, do_drain=(b >= L), do_fire=True)

        def group(g, carry):
            for b in range(M):
                visit(g * M + b, b, do_drain=True, do_fire=True)
            return carry

        lax.fori_loop(1, n_chunks // M - 1, group, 0)
        for b in range(M):
            c = n_chunks - M + b
            visit(c, b, do_drain=True, do_fire=(b < M - L))
        for b in range(L):
            c = n_chunks - L + b
            wait_s(c, c % M)


def _sc_mp1_body(hs_hbm, src_hbm, dst_hbm, zeros_hbm, out_hbm,
                 src_v, dst_v, rows_v, gsem, ssem, shared):
    cid = lax.axis_index("c")
    sid = lax.axis_index("s")
    wid = cid * 16 + sid
    n_chunks = EPAD // 32 // CHUNK

    pltpu.sync_copy(zeros_hbm.at[pl.ds(sid * ROWS_PER_SUB, ROWS_PER_SUB)],
                    shared.at[pl.ds(sid * ROWS_PER_SUB, ROWS_PER_SUB)])

    def load_idx(p):
        pltpu.sync_copy(src_hbm.at[wid], src_v)
        pltpu.sync_copy(dst_hbm.at[wid], dst_v)
        plsc.subcore_barrier()

    _sc_mp_pipeline(hs_hbm, src_v, dst_v, rows_v, gsem, ssem, shared,
                    n_chunks, 1, load_idx)

    plsc.subcore_barrier()
    pltpu.sync_copy(shared.at[pl.ds(sid * ROWS_PER_SUB, ROWS_PER_SUB)],
                    out_hbm.at[cid, pl.ds(sid * ROWS_PER_SUB, ROWS_PER_SUB)])


_sc_mp1 = pl.kernel(
    _sc_mp1_body,
    out_type=jax.ShapeDtypeStruct((2, NPAD, 64), jnp.bfloat16),
    mesh=_mesh,
    scratch_types=[
        pltpu.VMEM((EPAD // 32 // CHUNK, CHUNK), jnp.int32),
        pltpu.VMEM((EPAD // 32 // CHUNK, CHUNK), jnp.int32),
        pltpu.VMEM((M, CHUNK, 64), jnp.bfloat16),
        pltpu.SemaphoreType.DMA((M,)),
        pltpu.SemaphoreType.DMA((M,)),
        pltpu.VMEM_SHARED((NPAD, 64), jnp.bfloat16),
    ],
    compiler_params=pltpu.CompilerParams(use_tc_tiling_on_sc=False),
)


N_PHASES = 2
PHASE_CHUNKS = EPAD // 16 // CHUNK // N_PHASES


def _sc_mp2_body(hs_hbm, src_hbm, dst_hbm, zeros_hbm, out_hbm,
                 src_v, dst_v, rows_v, gsem, ssem, shared):
    cid = lax.axis_index("c")
    sid = lax.axis_index("s")
    table = hs_hbm.at[cid]

    pltpu.sync_copy(zeros_hbm.at[pl.ds(sid * ROWS_PER_SUB, ROWS_PER_SUB)],
                    shared.at[pl.ds(sid * ROWS_PER_SUB, ROWS_PER_SUB)])

    def load_idx(p):
        pltpu.sync_copy(src_hbm.at[2 * sid + p], src_v)
        pltpu.sync_copy(dst_hbm.at[2 * sid + p], dst_v)
        plsc.subcore_barrier()

    _sc_mp_pipeline(table, src_v, dst_v, rows_v, gsem, ssem, shared,
                    PHASE_CHUNKS, N_PHASES, load_idx)

    plsc.subcore_barrier()
    pltpu.sync_copy(shared.at[pl.ds(sid * ROWS_PER_SUB, ROWS_PER_SUB)],
                    out_hbm.at[cid, pl.ds(sid * ROWS_PER_SUB, ROWS_PER_SUB)])


_sc_mp2 = pl.kernel(
    _sc_mp2_body,
    out_type=jax.ShapeDtypeStruct((2, NPAD, 64), jnp.bfloat16),
    mesh=_mesh,
    scratch_types=[
        pltpu.VMEM((PHASE_CHUNKS, CHUNK), jnp.int32),
        pltpu.VMEM((PHASE_CHUNKS, CHUNK), jnp.int32),
        pltpu.VMEM((M, CHUNK, 64), jnp.bfloat16),
        pltpu.SemaphoreType.DMA((M,)),
        pltpu.SemaphoreType.DMA((M,)),
        pltpu.VMEM_SHARED((NPAD, 64), jnp.bfloat16),
    ],
    compiler_params=pltpu.CompilerParams(use_tc_tiling_on_sc=False),
)


def _dot1_body(x_ref, w1_ref, h_ref):
    h_ref[pl.ds(0, N_NODES), :] = jnp.dot(
        x_ref[...], w1_ref[...], preferred_element_type=jnp.float32)
    h_ref[pl.ds(N_NODES, NPAD - N_NODES), :] = jnp.zeros(
        (NPAD - N_NODES, 64), jnp.float32)


def _scale1_body(h_ref, degp_ref, hs1_ref, dinv_ref):
    degp = degp_ref[...]
    deg = 1.0 + degp[0, :, 0:1] + degp[1, :, 0:1]
    dinv = lax.rsqrt(deg)
    hs1_ref[...] = (h_ref[...] * dinv).astype(jnp.bfloat16)
    dinv_ref[...] = dinv


def _mm2_body(p1_ref, hs1_ref, dinv_ref, b1_ref, w2_ref, hs2_ref):
    p1 = p1_ref[...].astype(jnp.float32)
    hs1 = hs1_ref[...].astype(jnp.float32)
    dinv = dinv_ref[...]
    z = jax.nn.relu(dinv * (p1[0] + p1[1] + hs1) + b1_ref[...])
    hs2 = ((jnp.dot(z, w2_ref[...], preferred_element_type=jnp.float32))
           * dinv).astype(jnp.bfloat16)
    hs2_ref[0] = hs2[:, :64]
    hs2_ref[1] = hs2[:, 64:]


def _pool_body(p2_ref, hs2_ref, dinv_ref, b2_ref, batch_ref, out_ref):
    p2 = p2_ref[...].astype(jnp.float32)
    hs2 = hs2_ref[...].astype(jnp.float32)
    h2 = dinv_ref[...] * jnp.concatenate(
        [p2[0] + hs2[0], p2[1] + hs2[1]], axis=1)
    gids = lax.broadcasted_iota(jnp.int32, (N_GRAPHS, NPAD), 0)
    onehot = (gids == batch_ref[...]).astype(jnp.float32)
    pooled = jnp.dot(onehot, h2, preferred_element_type=jnp.float32)
    counts = jnp.maximum(jnp.sum(onehot, axis=1, keepdims=True), 1.0)
    out_ref[...] = pooled / counts + b2_ref[...]


_tc_dot1 = pl.pallas_call(
    _dot1_body,
    out_shape=jax.ShapeDtypeStruct((NPAD, 64), jnp.float32),
)

_tc_scale1 = pl.pallas_call(
    _scale1_body,
    out_shape=(jax.ShapeDtypeStruct((NPAD, 64), jnp.bfloat16),
               jax.ShapeDtypeStruct((NPAD, 1), jnp.float32)),
)

_tc_mm2 = pl.pallas_call(
    _mm2_body,
    out_shape=jax.ShapeDtypeStruct((2, NPAD, 64), jnp.bfloat16),
)

_tc_pool = pl.pallas_call(
    _pool_body,
    out_shape=jax.ShapeDtypeStruct((N_GRAPHS, 128), jnp.float32),
)


@jax.jit
def kernel(x, edge_index, batch, W1, b1, W2, b2):
    ei = edge_index.astype(jnp.int32)
    pad = N_NODES + (jnp.arange(EPAD - N_EDGES, dtype=jnp.int32)
                     % (NPAD - N_NODES))
    srcp = jnp.concatenate([ei[0], pad])
    dstp = jnp.concatenate([ei[1], pad])
    src32 = srcp.reshape(32, EPAD // 32 // CHUNK, CHUNK)
    dst32 = dstp.reshape(32, EPAD // 32 // CHUNK, CHUNK)
    batchp = jnp.pad(batch.astype(jnp.int32), (0, NPAD - N_NODES),
                     constant_values=-1).reshape(1, NPAD)

    zeros16 = jnp.zeros((NPAD, 16), jnp.float32)
    zeros64 = jnp.zeros((NPAD, 64), jnp.bfloat16)
    ones16 = jnp.ones((CHUNK, 16), jnp.float32)

    h = _tc_dot1(x.astype(jnp.float32), W1)
    degp = _sc_deg(dst32, zeros16, ones16)
    hs1, dinv = _tc_scale1(h, degp)
    p1 = _sc_mp1(hs1, src32, dst32, zeros64)
    hs2 = _tc_mm2(p1, hs1, dinv, b1.reshape(1, 64), W2)
    p2 = _sc_mp2(hs2, src32, dst32, zeros64)
    return _tc_pool(p2, hs2, dinv, b2.reshape(1, 128), batchp)


# speedup vs baseline: 1.0207x; 1.0175x over previous
import functools

import jax
import jax.numpy as jnp
from jax import lax
from jax.experimental import pallas as pl
from jax.experimental.pallas import tpu as pltpu
from jax.experimental.pallas import tpu_sc as plsc

N_NODES = 10000
N_EDGES = 320000
N_GRAPHS = 16

NPAD = 10112
EPAD = 327680
CHUNK = 128
ROWS_PER_SUB = NPAD // 16

M = 8
L = 4

_mesh = plsc.VectorSubcoreMesh(core_axis_name="c", subcore_axis_name="s")


def _sc_deg_body(dst_hbm, zeros_hbm, ones_hbm, out_hbm, dst_v, ones_v, sem,
                 shared):
    cid = lax.axis_index("c")
    sid = lax.axis_index("s")
    wid = cid * 16 + sid
    n_chunks = EPAD // 32 // CHUNK

    pltpu.sync_copy(zeros_hbm.at[pl.ds(sid * ROWS_PER_SUB, ROWS_PER_SUB)],
                    shared.at[pl.ds(sid * ROWS_PER_SUB, ROWS_PER_SUB)])
    pltpu.sync_copy(dst_hbm.at[wid], dst_v)
    pltpu.sync_copy(ones_hbm, ones_v)
    plsc.subcore_barrier()

    def fire(c, carry):
        pltpu.async_copy(ones_v, shared.at[dst_v.at[c]], sem, add=True)
        return carry

    lax.fori_loop(0, n_chunks, fire, 0)

    def drain(c, carry):
        pltpu.make_async_copy(ones_v, shared.at[dst_v.at[c]], sem).wait()
        return carry

    lax.fori_loop(0, n_chunks, drain, 0)
    plsc.subcore_barrier()
    pltpu.sync_copy(shared.at[pl.ds(sid * ROWS_PER_SUB, ROWS_PER_SUB)],
                    out_hbm.at[cid, pl.ds(sid * ROWS_PER_SUB, ROWS_PER_SUB)])


_sc_deg = pl.kernel(
    _sc_deg_body,
    out_type=jax.ShapeDtypeStruct((2, NPAD, 16), jnp.float32),
    mesh=_mesh,
    scratch_types=[
        pltpu.VMEM((EPAD // 32 // CHUNK, CHUNK), jnp.int32),
        pltpu.VMEM((CHUNK, 16), jnp.float32),
        pltpu.SemaphoreType.DMA,
        pltpu.VMEM_SHARED((NPAD, 16), jnp.float32),
    ],
    compiler_params=pltpu.CompilerParams(use_tc_tiling_on_sc=False),
)


def _sc_mp_pipeline(table, src_v, dst_v, rows_v, gsem, ssem, shared,
                    n_chunks, n_phases, load_idx):
    def fire_g(c, b):
        pltpu.async_copy(table.at[src_v.at[c]], rows_v.at[b], gsem.at[b])

    def wait_g(c, b):
        pltpu.make_async_copy(table.at[src_v.at[c]], rows_v.at[b],
                              gsem.at[b]).wait()

    def fire_s(c, b):
        pltpu.async_copy(rows_v.at[b], shared.at[dst_v.at[c]], ssem.at[b],
                         add=True)

    def wait_s(c, b):
        pltpu.make_async_copy(rows_v.at[b], shared.at[dst_v.at[c]],
                              ssem.at[b]).wait()

    def visit(c, b, do_drain, do_fire):
        wait_g(c, b)
        fire_s(c, b)
        cp = c + L
        if do_drain:
            wait_s(cp - M, cp % M)
        if do_fire:
            fire_g(cp, cp % M)

    for p in range(n_phases):
        load_idx(p)

        for b in range(L):
            fire_g(b, b)
        for b in range(M):
            visit(b, b, do_drain=(b >= L), do_fire=True)

        def group(g, carry):
            for b in range(M):
                visit(g * M + b, b, do_drain=True, do_fire=True)
            return carry

        lax.fori_loop(1, n_chunks // M - 1, group, 0)
        for b in range(M):
            c = n_chunks - M + b
            visit(c, b, do_drain=True, do_fire=(b < M - L))
        for b in range(L):
            c = n_chunks - L + b
            wait_s(c, c % M)


def _sc_mp1_body(hs_hbm, src_hbm, dst_hbm, zeros_hbm, out_hbm,
                 src_v, dst_v, rows_v, gsem, ssem, shared):
    cid = lax.axis_index("c")
    sid = lax.axis_index("s")
    wid = cid * 16 + sid
    n_chunks = EPAD // 32 // CHUNK

    pltpu.sync_copy(zeros_hbm.at[pl.ds(sid * ROWS_PER_SUB, ROWS_PER_SUB)],
                    shared.at[pl.ds(sid * ROWS_PER_SUB, ROWS_PER_SUB)])

    def load_idx(p):
        pltpu.sync_copy(src_hbm.at[wid], src_v)
        pltpu.sync_copy(dst_hbm.at[wid], dst_v)
        plsc.subcore_barrier()

    _sc_mp_pipeline(hs_hbm, src_v, dst_v, rows_v, gsem, ssem, shared,
                    n_chunks, 1, load_idx)

    plsc.subcore_barrier()
    pltpu.sync_copy(shared.at[pl.ds(sid * ROWS_PER_SUB, ROWS_PER_SUB)],
                    out_hbm.at[cid, pl.ds(sid * ROWS_PER_SUB, ROWS_PER_SUB)])


_sc_mp1 = pl.kernel(
    _sc_mp1_body,
    out_type=jax.ShapeDtypeStruct((2, NPAD, 64), jnp.bfloat16),
    mesh=_mesh,
    scratch_types=[
        pltpu.VMEM((EPAD // 32 // CHUNK, CHUNK), jnp.int32),
        pltpu.VMEM((EPAD // 32 // CHUNK, CHUNK), jnp.int32),
        pltpu.VMEM((M, CHUNK, 64), jnp.bfloat16),
        pltpu.SemaphoreType.DMA((M,)),
        pltpu.SemaphoreType.DMA((M,)),
        pltpu.VMEM_SHARED((NPAD, 64), jnp.bfloat16),
    ],
    compiler_params=pltpu.CompilerParams(use_tc_tiling_on_sc=False),
)


N_PHASES = 2
PHASE_CHUNKS = EPAD // 16 // CHUNK // N_PHASES


def _sc_mp2_body(hs_hbm, src_hbm, dst_hbm, zeros_hbm, out_hbm,
                 src_v, dst_v, rows_v, gsem, ssem, shared):
    cid = lax.axis_index("c")
    sid = lax.axis_index("s")
    table = hs_hbm.at[cid]

    pltpu.sync_copy(zeros_hbm.at[pl.ds(sid * ROWS_PER_SUB, ROWS_PER_SUB)],
                    shared.at[pl.ds(sid * ROWS_PER_SUB, ROWS_PER_SUB)])

    def load_idx(p):
        pltpu.sync_copy(src_hbm.at[2 * sid + p], src_v)
        pltpu.sync_copy(dst_hbm.at[2 * sid + p], dst_v)
        plsc.subcore_barrier()

    _sc_mp_pipeline(table, src_v, dst_v, rows_v, gsem, ssem, shared,
                    PHASE_CHUNKS, N_PHASES, load_idx)

    plsc.subcore_barrier()
    pltpu.sync_copy(shared.at[pl.ds(sid * ROWS_PER_SUB, ROWS_PER_SUB)],
                    out_hbm.at[cid, pl.ds(sid * ROWS_PER_SUB, ROWS_PER_SUB)])


_sc_mp2 = pl.kernel(
    _sc_mp2_body,
    out_type=jax.ShapeDtypeStruct((2, NPAD, 64), jnp.bfloat16),
    mesh=_mesh,
    scratch_types=[
        pltpu.VMEM((PHASE_CHUNKS, CHUNK), jnp.int32),
        pltpu.VMEM((PHASE_CHUNKS, CHUNK), jnp.int32),
        pltpu.VMEM((M, CHUNK, 64), jnp.bfloat16),
        pltpu.SemaphoreType.DMA((M,)),
        pltpu.SemaphoreType.DMA((M,)),
        pltpu.VMEM_SHARED((NPAD, 64), jnp.bfloat16),
    ],
    compiler_params=pltpu.CompilerParams(use_tc_tiling_on_sc=False),
)


def _dot1_body(x_ref, w1_ref, h_ref):
    h_ref[pl.ds(0, N_NODES), :] = jnp.dot(
        x_ref[...], w1_ref[...], preferred_element_type=jnp.float32)
    h_ref[pl.ds(N_NODES, NPAD - N_NODES), :] = jnp.zeros(
        (NPAD - N_NODES, 64), jnp.float32)


def _scale1_body(h_ref, degp_ref, hs1_ref, dinv_ref):
    degp = degp_ref[...]
    deg = 1.0 + degp[0, :, 0:1] + degp[1, :, 0:1]
    dinv = lax.rsqrt(deg)
    hs1_ref[...] = (h_ref[...] * dinv).astype(jnp.bfloat16)
    dinv_ref[...] = dinv


def _mm2_body(p1_ref, hs1_ref, dinv_ref, b1_ref, w2_ref, hs2_ref):
    p1 = p1_ref[...].astype(jnp.float32)
    hs1 = hs1_ref[...].astype(jnp.float32)
    dinv = dinv_ref[...]
    z = jax.nn.relu(dinv * (p1[0] + p1[1] + hs1) + b1_ref[...])
    hs2 = ((jnp.dot(z, w2_ref[...], preferred_element_type=jnp.float32))
           * dinv).astype(jnp.bfloat16)
    hs2_ref[0] = hs2[:, :64]
    hs2_ref[1] = hs2[:, 64:]


def _pool_body(p2_ref, hs2_ref, dinv_ref, b2_ref, batch_ref, out_ref):
    p2 = p2_ref[...].astype(jnp.float32)
    hs2 = hs2_ref[...].astype(jnp.float32)
    h2 = dinv_ref[...] * jnp.concatenate(
        [p2[0] + hs2[0], p2[1] + hs2[1]], axis=1)
    gids = lax.broadcasted_iota(jnp.int32, (N_GRAPHS, NPAD), 0)
    onehot = (gids == batch_ref[...]).astype(jnp.float32)
    pooled = jnp.dot(onehot, h2, preferred_element_type=jnp.float32)
    counts = jnp.maximum(jnp.sum(onehot, axis=1, keepdims=True), 1.0)
    out_ref[...] = pooled / counts + b2_ref[...]


_tc_dot1 = pl.pallas_call(
    _dot1_body,
    out_shape=jax.ShapeDtypeStruct((NPAD, 64), jnp.float32),
)

_tc_scale1 = pl.pallas_call(
    _scale1_body,
    out_shape=(jax.ShapeDtypeStruct((NPAD, 64), jnp.bfloat16),
               jax.ShapeDtypeStruct((NPAD, 1), jnp.float32)),
)

_tc_mm2 = pl.pallas_call(
    _mm2_body,
    out_shape=jax.ShapeDtypeStruct((2, NPAD, 64), jnp.bfloat16),
)

_tc_pool = pl.pallas_call(
    _pool_body,
    out_shape=jax.ShapeDtypeStruct((N_GRAPHS, 128), jnp.float32),
)


@jax.jit
def kernel(x, edge_index, batch, W1, b1, W2, b2):
    ei = edge_index.astype(jnp.int32)
    pad = N_NODES + (jnp.arange(EPAD - N_EDGES, dtype=jnp.int32)
                     % (NPAD - N_NODES))
    srcp = jnp.concatenate([ei[0], pad])
    dstp = jnp.concatenate([ei[1], pad])
    src32 = srcp.reshape(32, EPAD // 32 // CHUNK, CHUNK)
    dst32 = dstp.reshape(32, EPAD // 32 // CHUNK, CHUNK)
    batchp = jnp.pad(batch.astype(jnp.int32), (0, NPAD - N_NODES),
                     constant_values=-1).reshape(1, NPAD)

    zeros16 = jnp.zeros((NPAD, 16), jnp.float32)
    zeros64 = jnp.zeros((NPAD, 64), jnp.bfloat16)
    ones16 = jnp.ones((CHUNK, 16), jnp.float32)

    h = _tc_dot1(x.astype(jnp.float32), W1)
    degp = _sc_deg(dst32, zeros16, ones16)
    hs1, dinv = _tc_scale1(h, degp)
    p1 = _sc_mp1(hs1, src32, dst32, zeros64)
    hs2 = _tc_mm2(p1, hs1, dinv, b1.reshape(1, 64), W2)
    p2 = _sc_mp2(hs2, src32, dst32, zeros64)
    return _tc_pool(p2, hs2, dinv, b2.reshape(1, 128), batchp)


# speedup vs baseline: 1.0353x; 1.0142x over previous
import functools

import jax
import jax.numpy as jnp
from jax import lax
from jax.experimental import pallas as pl
from jax.experimental.pallas import tpu as pltpu
from jax.experimental.pallas import tpu_sc as plsc

N_NODES = 10000
N_EDGES = 320000
N_GRAPHS = 16

NPAD = 10112
EPAD = 327680
CHUNK = 128
ROWS_PER_SUB = NPAD // 16

M = 8
L = 4

_mesh = plsc.VectorSubcoreMesh(core_axis_name="c", subcore_axis_name="s")


def _sc_deg_body(dst_hbm, zeros_hbm, ones_hbm, out_hbm, dst_v, ones_v, sem,
                 shared):
    cid = lax.axis_index("c")
    sid = lax.axis_index("s")
    wid = cid * 16 + sid
    n_chunks = EPAD // 32 // CHUNK

    pltpu.sync_copy(zeros_hbm.at[pl.ds(sid * ROWS_PER_SUB, ROWS_PER_SUB)],
                    shared.at[pl.ds(sid * ROWS_PER_SUB, ROWS_PER_SUB)])
    pltpu.sync_copy(dst_hbm.at[wid], dst_v)
    pltpu.sync_copy(ones_hbm, ones_v)
    plsc.subcore_barrier()

    def fire(c, carry):
        pltpu.async_copy(ones_v, shared.at[dst_v.at[c]], sem, add=True)
        return carry

    lax.fori_loop(0, n_chunks, fire, 0)

    def drain(c, carry):
        pltpu.make_async_copy(ones_v, shared.at[dst_v.at[c]], sem).wait()
        return carry

    lax.fori_loop(0, n_chunks, drain, 0)
    plsc.subcore_barrier()
    pltpu.sync_copy(shared.at[pl.ds(sid * ROWS_PER_SUB, ROWS_PER_SUB)],
                    out_hbm.at[cid, pl.ds(sid * ROWS_PER_SUB, ROWS_PER_SUB)])


_sc_deg = pl.kernel(
    _sc_deg_body,
    out_type=jax.ShapeDtypeStruct((2, NPAD, 8), jnp.float32),
    mesh=_mesh,
    scratch_types=[
        pltpu.VMEM((EPAD // 32 // CHUNK, CHUNK), jnp.int32),
        pltpu.VMEM((CHUNK, 8), jnp.float32),
        pltpu.SemaphoreType.DMA,
        pltpu.VMEM_SHARED((NPAD, 8), jnp.float32),
    ],
    compiler_params=pltpu.CompilerParams(use_tc_tiling_on_sc=False),
)


def _sc_mp_pipeline(table, src_v, dst_v, rows_v, gsem, ssem, shared,
                    n_chunks, n_phases, load_idx):
    def fire_g(c, b):
        pltpu.async_copy(table.at[src_v.at[c]], rows_v.at[b], gsem.at[b])

    def wait_g(c, b):
        pltpu.make_async_copy(table.at[src_v.at[c]], rows_v.at[b],
                              gsem.at[b]).wait()

    def fire_s(c, b):
        pltpu.async_copy(rows_v.at[b], shared.at[dst_v.at[c]], ssem.at[b],
                         add=True)

    def wait_s(c, b):
        pltpu.make_async_copy(rows_v.at[b], shared.at[dst_v.at[c]],
                              ssem.at[b]).wait()

    def visit(c, b, do_drain, do_fire):
        wait_g(c, b)
        fire_s(c, b)
        cp = c + L
        if do_drain:
            wait_s(cp - M, cp % M)
        if do_fire:
            fire_g(cp, cp % M)

    for p in range(n_phases):
        load_idx(p)

        for b in range(L):
            fire_g(b, b)
        for b in range(M):
            visit(b, b, do_drain=(b >= L), do_fire=True)

        def group(g, carry):
            for b in range(M):
                visit(g * M + b, b, do_drain=True, do_fire=True)
            return carry

        lax.fori_loop(1, n_chunks // M - 1, group, 0)
        for b in range(M):
            c = n_chunks - M + b
            visit(c, b, do_drain=True, do_fire=(b < M - L))
        for b in range(L):
            c = n_chunks - L + b
            wait_s(c, c % M)


def _sc_mp1_body(hs_hbm, src_hbm, dst_hbm, zeros_hbm, out_hbm,
                 src_v, dst_v, rows_v, gsem, ssem, shared):
    cid = lax.axis_index("c")
    sid = lax.axis_index("s")
    wid = cid * 16 + sid
    n_chunks = EPAD // 32 // CHUNK

    pltpu.sync_copy(zeros_hbm.at[pl.ds(sid * ROWS_PER_SUB, ROWS_PER_SUB)],
                    shared.at[pl.ds(sid * ROWS_PER_SUB, ROWS_PER_SUB)])

    def load_idx(p):
        pltpu.sync_copy(src_hbm.at[wid], src_v)
        pltpu.sync_copy(dst_hbm.at[wid], dst_v)
        plsc.subcore_barrier()

    _sc_mp_pipeline(hs_hbm, src_v, dst_v, rows_v, gsem, ssem, shared,
                    n_chunks, 1, load_idx)

    plsc.subcore_barrier()
    pltpu.sync_copy(shared.at[pl.ds(sid * ROWS_PER_SUB, ROWS_PER_SUB)],
                    out_hbm.at[cid, pl.ds(sid * ROWS_PER_SUB, ROWS_PER_SUB)])


_sc_mp1 = pl.kernel(
    _sc_mp1_body,
    out_type=jax.ShapeDtypeStruct((2, NPAD, 64), jnp.bfloat16),
    mesh=_mesh,
    scratch_types=[
        pltpu.VMEM((EPAD // 32 // CHUNK, CHUNK), jnp.int32),
        pltpu.VMEM((EPAD // 32 // CHUNK, CHUNK), jnp.int32),
        pltpu.VMEM((M, CHUNK, 64), jnp.bfloat16),
        pltpu.SemaphoreType.DMA((M,)),
        pltpu.SemaphoreType.DMA((M,)),
        pltpu.VMEM_SHARED((NPAD, 64), jnp.bfloat16),
    ],
    compiler_params=pltpu.CompilerParams(use_tc_tiling_on_sc=False),
)


N_PHASES = 2
PHASE_CHUNKS = EPAD // 16 // CHUNK // N_PHASES


def _sc_mp2_body(hs_hbm, src_hbm, dst_hbm, zeros_hbm, out_hbm,
                 src_v, dst_v, rows_v, gsem, ssem, shared):
    cid = lax.axis_index("c")
    sid = lax.axis_index("s")
    table = hs_hbm.at[cid]

    pltpu.sync_copy(zeros_hbm.at[pl.ds(sid * ROWS_PER_SUB, ROWS_PER_SUB)],
                    shared.at[pl.ds(sid * ROWS_PER_SUB, ROWS_PER_SUB)])

    def load_idx(p):
        pltpu.sync_copy(src_hbm.at[2 * sid + p], src_v)
        pltpu.sync_copy(dst_hbm.at[2 * sid + p], dst_v)
        plsc.subcore_barrier()

    _sc_mp_pipeline(table, src_v, dst_v, rows_v, gsem, ssem, shared,
                    PHASE_CHUNKS, N_PHASES, load_idx)

    plsc.subcore_barrier()
    pltpu.sync_copy(shared.at[pl.ds(sid * ROWS_PER_SUB, ROWS_PER_SUB)],
                    out_hbm.at[cid, pl.ds(sid * ROWS_PER_SUB, ROWS_PER_SUB)])


_sc_mp2 = pl.kernel(
    _sc_mp2_body,
    out_type=jax.ShapeDtypeStruct((2, NPAD, 64), jnp.bfloat16),
    mesh=_mesh,
    scratch_types=[
        pltpu.VMEM((PHASE_CHUNKS, CHUNK), jnp.int32),
        pltpu.VMEM((PHASE_CHUNKS, CHUNK), jnp.int32),
        pltpu.VMEM((M, CHUNK, 64), jnp.bfloat16),
        pltpu.SemaphoreType.DMA((M,)),
        pltpu.SemaphoreType.DMA((M,)),
        pltpu.VMEM_SHARED((NPAD, 64), jnp.bfloat16),
    ],
    compiler_params=pltpu.CompilerParams(use_tc_tiling_on_sc=False),
)


def _dot1_body(x_ref, w1_ref, h_ref):
    h_ref[pl.ds(0, N_NODES), :] = jnp.dot(
        x_ref[...], w1_ref[...], preferred_element_type=jnp.float32)
    h_ref[pl.ds(N_NODES, NPAD - N_NODES), :] = jnp.zeros(
        (NPAD - N_NODES, 64), jnp.float32)


def _scale1_body(h_ref, degp_ref, hs1_ref, dinv_ref):
    degp = degp_ref[...]
    deg = 1.0 + degp[0, :, 0:1] + degp[1, :, 0:1]
    dinv = lax.rsqrt(deg)
    hs1_ref[...] = (h_ref[...] * dinv).astype(jnp.bfloat16)
    dinv_ref[...] = dinv


def _mm2_body(p1_ref, hs1_ref, dinv_ref, b1_ref, w2_ref, hs2_ref):
    p1 = p1_ref[...].astype(jnp.float32)
    hs1 = hs1_ref[...].astype(jnp.float32)
    dinv = dinv_ref[...]
    z = jax.nn.relu(dinv * (p1[0] + p1[1] + hs1) + b1_ref[...])
    hs2 = ((jnp.dot(z, w2_ref[...], preferred_element_type=jnp.float32))
           * dinv).astype(jnp.bfloat16)
    hs2_ref[0] = hs2[:, :64]
    hs2_ref[1] = hs2[:, 64:]


def _pool_body(p2_ref, hs2_ref, dinv_ref, b2_ref, batch_ref, out_ref):
    p2 = p2_ref[...].astype(jnp.float32)
    hs2 = hs2_ref[...].astype(jnp.float32)
    h2 = dinv_ref[...] * jnp.concatenate(
        [p2[0] + hs2[0], p2[1] + hs2[1]], axis=1)
    gids = lax.broadcasted_iota(jnp.int32, (N_GRAPHS, NPAD), 0)
    onehot = (gids == batch_ref[...]).astype(jnp.float32)
    pooled = jnp.dot(onehot, h2, preferred_element_type=jnp.float32)
    counts = jnp.maximum(jnp.sum(onehot, axis=1, keepdims=True), 1.0)
    out_ref[...] = pooled / counts + b2_ref[...]


_tc_dot1 = pl.pallas_call(
    _dot1_body,
    out_shape=jax.ShapeDtypeStruct((NPAD, 64), jnp.float32),
)

_tc_scale1 = pl.pallas_call(
    _scale1_body,
    out_shape=(jax.ShapeDtypeStruct((NPAD, 64), jnp.bfloat16),
               jax.ShapeDtypeStruct((NPAD, 1), jnp.float32)),
)

_tc_mm2 = pl.pallas_call(
    _mm2_body,
    out_shape=jax.ShapeDtypeStruct((2, NPAD, 64), jnp.bfloat16),
)

_tc_pool = pl.pallas_call(
    _pool_body,
    out_shape=jax.ShapeDtypeStruct((N_GRAPHS, 128), jnp.float32),
)


@jax.jit
def kernel(x, edge_index, batch, W1, b1, W2, b2):
    ei = edge_index.astype(jnp.int32)
    pad = N_NODES + (jnp.arange(EPAD - N_EDGES, dtype=jnp.int32)
                     % (NPAD - N_NODES))
    srcp = jnp.concatenate([ei[0], pad])
    dstp = jnp.concatenate([ei[1], pad])
    src32 = srcp.reshape(32, EPAD // 32 // CHUNK, CHUNK)
    dst32 = dstp.reshape(32, EPAD // 32 // CHUNK, CHUNK)
    batchp = jnp.pad(batch.astype(jnp.int32), (0, NPAD - N_NODES),
                     constant_values=-1).reshape(1, NPAD)

    zeros16 = jnp.zeros((NPAD, 8), jnp.float32)
    zeros64 = jnp.zeros((NPAD, 64), jnp.bfloat16)
    ones16 = jnp.ones((CHUNK, 8), jnp.float32)

    h = _tc_dot1(x.astype(jnp.float32), W1)
    degp = _sc_deg(dst32, zeros16, ones16)
    hs1, dinv = _tc_scale1(h, degp)
    p1 = _sc_mp1(hs1, src32, dst32, zeros64)
    hs2 = _tc_mm2(p1, hs1, dinv, b1.reshape(1, 64), W2)
    p2 = _sc_mp2(hs2, src32, dst32, zeros64)
    return _tc_pool(p2, hs2, dinv, b2.reshape(1, 128), batchp)
